# baseline (speedup 1.0000x reference)
import functools

import jax
import jax.numpy as jnp
from jax.experimental import pallas as pl
from jax.experimental.pallas import tpu as pltpu

BM = 1024
BK = 4096


def _fused_body(att_ref, x_ref, w_ref, a_ref, b_ref, o_ref, s_ref,
                *, num_rel, num_k):
    r = pl.program_id(1)
    k = pl.program_id(2)

    @pl.when((r == 0) & (k == 0))
    def _compute_support():
        m = att_ref[0]
        for j in range(1, num_rel):
            m = jnp.maximum(m, att_ref[j])
        denom = jnp.exp(att_ref[0] - m)
        for j in range(1, num_rel):
            denom = denom + jnp.exp(att_ref[j] - m)
        x = x_ref[...]
        for j in range(num_rel):
            att_j = jnp.exp(att_ref[j] - m) / denom
            s_ref[j] = (jnp.dot(x, w_ref[j], preferred_element_type=jnp.float32)
                        * att_j).astype(jnp.bfloat16)
        o_ref[...] = jnp.broadcast_to(b_ref[...], o_ref.shape)

    o_ref[...] += jnp.dot(a_ref[0].astype(jnp.bfloat16),
                          s_ref[r, pl.ds(k * BK, BK), :],
                          preferred_element_type=jnp.float32)


def kernel(input, adjs, adj_weight, attention, bias):
    num_rel, n, _ = adjs.shape
    d_in = input.shape[1]
    d_out = adj_weight.shape[2]
    num_k = n // BK

    out = pl.pallas_call(
        functools.partial(_fused_body, num_rel=num_rel, num_k=num_k),
        grid=(n // BM, num_rel, num_k),
        in_specs=[
            pl.BlockSpec(memory_space=pltpu.SMEM),
            pl.BlockSpec((n, d_in), lambda i, r, k: (0, 0)),
            pl.BlockSpec((num_rel, d_in, d_out), lambda i, r, k: (0, 0, 0)),
            pl.BlockSpec((1, BM, BK), lambda i, r, k: (r, i, k)),
            pl.BlockSpec((1, d_out), lambda i, r, k: (0, 0)),
        ],
        out_specs=pl.BlockSpec((BM, d_out), lambda i, r, k: (i, 0)),
        out_shape=jax.ShapeDtypeStruct((n, d_out), jnp.float32),
        scratch_shapes=[pltpu.VMEM((num_rel, n, d_out), jnp.bfloat16)],
        compiler_params=pltpu.CompilerParams(
            dimension_semantics=("parallel", "arbitrary", "arbitrary"),
        ),
    )(attention, input, adj_weight, adjs, bias.reshape(1, d_out))
    return out


# speedup vs baseline: 1.0271x; 1.0271x over previous
import functools

import jax
import jax.numpy as jnp
from jax.experimental import pallas as pl
from jax.experimental.pallas import tpu as pltpu

BM = 4096
BK = 512


def _fused_body(att_ref, x_ref, w_ref, a_ref, b_ref, o_ref, s_ref,
                *, num_rel, num_k):
    r = pl.program_id(1)
    k = pl.program_id(2)

    @pl.when(k == 0)
    def _compute_support():
        m = att_ref[0]
        for j in range(1, num_rel):
            m = jnp.maximum(m, att_ref[j])
        denom = jnp.exp(att_ref[0] - m)
        for j in range(1, num_rel):
            denom = denom + jnp.exp(att_ref[j] - m)
        att_r = jnp.exp(att_ref[r] - m) / denom
        w = w_ref[r, pl.ds(0, w_ref.shape[1]), :]
        s_ref[r] = (jnp.dot(x_ref[...], w, preferred_element_type=jnp.float32)
                    * att_r).astype(jnp.bfloat16)

    @pl.when((r == 0) & (k == 0))
    def _init_out():
        o_ref[...] = jnp.broadcast_to(b_ref[...], o_ref.shape)

    o_ref[...] += jnp.dot(a_ref[0].astype(jnp.bfloat16),
                          s_ref[r, pl.ds(k * BK, BK), :],
                          preferred_element_type=jnp.float32)


def kernel(input, adjs, adj_weight, attention, bias):
    num_rel, n, _ = adjs.shape
    d_in = input.shape[1]
    d_out = adj_weight.shape[2]
    num_k = n // BK

    out = pl.pallas_call(
        functools.partial(_fused_body, num_rel=num_rel, num_k=num_k),
        grid=(n // BM, num_rel, num_k),
        in_specs=[
            pl.BlockSpec(memory_space=pltpu.SMEM),
            pl.BlockSpec((n, d_in), lambda i, r, k: (0, 0)),
            pl.BlockSpec((num_rel, d_in, d_out), lambda i, r, k: (0, 0, 0)),
            pl.BlockSpec((1, BM, BK), lambda i, r, k: (r, i, k)),
            pl.BlockSpec((1, d_out), lambda i, r, k: (0, 0)),
        ],
        out_specs=pl.BlockSpec((BM, d_out), lambda i, r, k: (i, 0)),
        out_shape=jax.ShapeDtypeStruct((n, d_out), jnp.float32),
        scratch_shapes=[pltpu.VMEM((num_rel, n, d_out), jnp.bfloat16)],
        compiler_params=pltpu.CompilerParams(
            dimension_semantics=("parallel", "arbitrary", "arbitrary"),
        ),
    )(attention, input, adj_weight, adjs, bias.reshape(1, d_out))
    return out
